# Initial kernel scaffold; baseline (speedup 1.0000x reference)
import jax
import jax.numpy as jnp
from jax.experimental import pallas as pl


def kernel(y, edge_index, edge_weight, mask):
    raise NotImplementedError("write your pallas kernel here")


# speedup vs baseline: 4.0953x; 4.0953x over previous
import jax
import jax.numpy as jnp
from jax import lax
from jax.experimental import pallas as pl
from jax.experimental.pallas import tpu as pltpu
from jax.experimental.pallas import tpu_sc as plsc

N = 10000
E = 160000
C = 256
N_LAYERS = 3
ALPHA = 0.9

NC = 2
NS = 16
L = 16
CH = C // NC
NP = 10240
K = 128
SB = 8
EPT = E // NS
EPT_PAD = ((EPT + K - 1) // K) * K
NB = EPT_PAD // K
NSB = NB // SB
RPT = NP // NS
R = 32
NRB = RPT // R
ZR = 16


def _expand_scalars(dst, wvec_of_g, nrows):
    def body(g, _):
        wv = wvec_of_g(g)
        for l in range(L):
            dst[g * L + l] = jnp.full((L,), wv[l], jnp.float32)
        return 0
    lax.fori_loop(0, nrows // L, body, 0)


def _lpa_body(y2, maskf, src3, dst3, w3,
              out2, bufA, bufB, res_h,
              acc, src_v, dst_v, w_v, rowbuf, wbuf, rbuf, zbuf, mbuf, bexp):
    c = lax.axis_index("c")
    s = lax.axis_index("s")
    row0 = s * RPT
    NCH = CH // L

    def zbody(i, _):
        for cc in range(NCH):
            zbuf[i, pl.ds(cc * L, L)] = jnp.zeros((L,), jnp.float32)
        return 0
    lax.fori_loop(0, ZR, zbody, 0)

    def pro_body(b, _):
        r0 = row0 + b * R
        sl = pl.ds(r0, R)
        pltpu.sync_copy(y2.at[c].at[sl], wbuf)
        pltpu.sync_copy(maskf.at[sl], mbuf)
        _expand_scalars(bexp, lambda g: mbuf[pl.ds(g * L, L)], R)

        def mrow(i, _):
            mv = bexp[i]
            for cc in range(NCH):
                csl = pl.ds(cc * L, L)
                v = wbuf[i, csl] * mv
                wbuf[i, csl] = v
                rbuf[i, csl] = v * (1.0 - ALPHA)
            return 0
        lax.fori_loop(0, R, mrow, 0)
        pltpu.sync_copy(wbuf, bufA.at[c].at[sl])
        pltpu.sync_copy(rbuf, res_h.at[c].at[sl])
        for q in range(R // ZR):
            pltpu.sync_copy(zbuf, acc.at[pl.ds(r0 + q * ZR, ZR)])
        return 0
    lax.fori_loop(0, NRB, pro_body, 0)
    plsc.subcore_barrier()

    srcs = [bufA, bufB, bufA]
    dsts = [bufB, bufA, out2]
    for layer in range(N_LAYERS):
        cur = srcs[layer]
        nxt = dsts[layer]

        def sb_body(sb, _):
            esl = pl.ds(sb * SB, SB)
            pltpu.sync_copy(src3.at[s].at[esl], src_v)
            pltpu.sync_copy(dst3.at[s].at[esl], dst_v)
            pltpu.sync_copy(w3.at[s].at[esl], w_v)

            def batch_body(j, _):
                pltpu.sync_copy(cur.at[c].at[src_v.at[j]], rowbuf)
                _expand_scalars(bexp, lambda g: w_v[j, pl.ds(g * L, L)], K)

                def srow(i, _):
                    wv = bexp[i]
                    for cc in range(NCH):
                        csl = pl.ds(cc * L, L)
                        rowbuf[i, csl] = rowbuf[i, csl] * wv
                    return 0
                lax.fori_loop(0, K, srow, 0)
                pltpu.sync_copy(rowbuf, acc.at[dst_v.at[j]], add=True)
                return 0
            lax.fori_loop(0, SB, batch_body, 0)
            return 0
        lax.fori_loop(0, NSB, sb_body, 0)
        plsc.subcore_barrier()

        def wb_batch(b, _):
            r0 = row0 + b * R
            sl = pl.ds(r0, R)
            pltpu.sync_copy(acc.at[sl], wbuf)
            pltpu.sync_copy(res_h.at[c].at[sl], rbuf)

            def wb_row(i, _):
                for cc in range(NCH):
                    csl = pl.ds(cc * L, L)
                    v = wbuf[i, csl] * ALPHA + rbuf[i, csl]
                    wbuf[i, csl] = jnp.clip(v, 0.0, 1.0)
                return 0
            lax.fori_loop(0, R, wb_row, 0)
            pltpu.sync_copy(wbuf, nxt.at[c].at[sl])
            if layer < N_LAYERS - 1:
                for q in range(R // ZR):
                    pltpu.sync_copy(zbuf, acc.at[pl.ds(r0 + q * ZR, ZR)])
            return 0
        lax.fori_loop(0, NRB, wb_batch, 0)
        plsc.subcore_barrier()


@jax.jit
def _lpa_call(y2, maskf, src3, dst3, w3):
    mesh = plsc.VectorSubcoreMesh(
        core_axis_name="c", subcore_axis_name="s",
        num_cores=NC, num_subcores=NS)
    f32 = jnp.float32
    out_types = (
        jax.ShapeDtypeStruct((NC, NP, CH), f32),
        jax.ShapeDtypeStruct((NC, NP, CH), f32),
        jax.ShapeDtypeStruct((NC, NP, CH), f32),
        jax.ShapeDtypeStruct((NC, NP, CH), f32),
    )
    scratch = [
        pltpu.VMEM_SHARED((NP, CH), f32),
        pltpu.VMEM((SB, K), jnp.int32),
        pltpu.VMEM((SB, K), jnp.int32),
        pltpu.VMEM((SB, K), f32),
        pltpu.VMEM((K, CH), f32),
        pltpu.VMEM((R, CH), f32),
        pltpu.VMEM((R, CH), f32),
        pltpu.VMEM((ZR, CH), f32),
        pltpu.VMEM((R,), f32),
        pltpu.VMEM((K, L), f32),
    ]
    fn = pl.kernel(_lpa_body, out_type=out_types, mesh=mesh,
                   scratch_types=scratch)
    return fn(y2, maskf, src3, dst3, w3)[0]


def kernel(y, edge_index, edge_weight, mask):
    y2 = y.reshape(N, NC, CH).transpose(1, 0, 2)
    y2 = jnp.pad(y2, ((0, 0), (0, NP - N), (0, 0)))
    maskf = jnp.pad(mask.astype(jnp.float32), (0, NP - N))
    src = edge_index[1].astype(jnp.int32).reshape(NS, EPT)
    dst = edge_index[0].astype(jnp.int32).reshape(NS, EPT)
    w = edge_weight.astype(jnp.float32).reshape(NS, EPT)
    pad = ((0, 0), (0, EPT_PAD - EPT))
    src3 = jnp.pad(src, pad).reshape(NS, NB, K)
    dst3 = jnp.pad(dst, pad).reshape(NS, NB, K)
    w3 = jnp.pad(w, pad).reshape(NS, NB, K)
    out2 = _lpa_call(y2, maskf, src3, dst3, w3)
    return out2.transpose(1, 0, 2).reshape(NP, C)[:N]
